# Initial kernel scaffold; baseline (speedup 1.0000x reference)
import jax
import jax.numpy as jnp
from jax.experimental import pallas as pl


def kernel(x, table, gamma, beta, W1, b1, W2, b2, W3, b3):
    raise NotImplementedError("write your pallas kernel here")


# speedup vs baseline: 10.3766x; 10.3766x over previous
import functools

import jax
import jax.numpy as jnp
from jax import lax
from jax.experimental import pallas as pl
from jax.experimental.pallas import tpu as pltpu
from jax.experimental.pallas import tpu_sc as plsc

B, F, L, V, D = 16384, 26, 20, 1000000, 16
H = F * D
BF = B * F
NC, NS = 2, 16
NW = NC * NS
SW = BF // NW
C = 32
CI = C * L
G = CI // 128
CHUNKS = SW // C
IDX_W = SW * L


def _pool_sc(x1d, table):
    mesh = plsc.VectorSubcoreMesh(core_axis_name="c", subcore_axis_name="s")

    @functools.partial(
        pl.kernel,
        out_type=jax.ShapeDtypeStruct((BF, D), jnp.float32),
        mesh=mesh,
        scratch_types=[
            pltpu.VMEM((CI,), jnp.int32),
            pltpu.VMEM((CI, D), jnp.float32),
            pltpu.VMEM((C, D), jnp.float32),
            pltpu.SemaphoreType.DMA,
        ],
        compiler_params=pltpu.CompilerParams(use_tc_tiling_on_sc=False),
    )
    def pool_kernel(x_hbm, tbl_hbm, out_hbm, idx_v, rows_v, pooled_v, sem):
        wid = lax.axis_index("s") * NC + lax.axis_index("c")

        @pl.loop(0, CHUNKS)
        def _chunk(g):
            i0 = wid * IDX_W + g * CI
            pltpu.sync_copy(x_hbm.at[pl.ds(i0, CI)], idx_v)
            copies = [
                pltpu.async_copy(
                    tbl_hbm.at[idx_v.at[pl.ds(j * 128, 128)]],
                    rows_v.at[pl.ds(j * 128, 128)],
                    sem,
                )
                for j in range(G)
            ]
            for cp in copies:
                cp.wait()

            @pl.loop(0, C)
            def _seg(s):
                r0 = s * L
                acc = rows_v[r0]
                for l in range(1, L):
                    acc = acc + rows_v[r0 + l]
                pooled_v[s] = acc

            pltpu.sync_copy(pooled_v, out_hbm.at[pl.ds(wid * SW + g * C, C)])

    return pool_kernel(x1d, table)


BB = 512


def _mlp_tc(sparse, gamma, beta, W1, b1, W2, b2, W3, b3):
    def body(p_ref, g_ref, be_ref, w1_ref, b1_ref, w2_ref, b2_ref, w3_ref,
             b3_ref, o_ref):
        sp = p_ref[...]
        mu = jnp.mean(sp, axis=-1, keepdims=True)
        var = jnp.mean((sp - mu) ** 2, axis=-1, keepdims=True)
        h = (sp - mu) / jnp.sqrt(var + 1e-5) * g_ref[...] + be_ref[...]
        h = jnp.maximum(
            jnp.dot(h, w1_ref[...], preferred_element_type=jnp.float32)
            + b1_ref[...], 0.0)
        h = jnp.maximum(
            jnp.dot(h, w2_ref[...], preferred_element_type=jnp.float32)
            + b2_ref[...], 0.0)
        o = jnp.dot(h, w3_ref[...], preferred_element_type=jnp.float32) + b3_ref[...]
        o_ref[...] = jax.nn.sigmoid(o)

    return pl.pallas_call(
        body,
        grid=(B // BB,),
        in_specs=[
            pl.BlockSpec((BB, H), lambda i: (i, 0)),
            pl.BlockSpec((H,), lambda i: (0,)),
            pl.BlockSpec((H,), lambda i: (0,)),
            pl.BlockSpec((H, 1024), lambda i: (0, 0)),
            pl.BlockSpec((1024,), lambda i: (0,)),
            pl.BlockSpec((1024, 512), lambda i: (0, 0)),
            pl.BlockSpec((512,), lambda i: (0,)),
            pl.BlockSpec((512, 1), lambda i: (0, 0)),
            pl.BlockSpec((1,), lambda i: (0,)),
        ],
        out_specs=pl.BlockSpec((BB, 1), lambda i: (i, 0)),
        out_shape=jax.ShapeDtypeStruct((B, 1), jnp.float32),
    )(sparse, gamma, beta, W1, b1, W2, b2, W3, b3)


def kernel(x, table, gamma, beta, W1, b1, W2, b2, W3, b3):
    x1d = x.reshape(-1)
    pooled = _pool_sc(x1d, table)
    sparse = pooled.reshape(B, H)
    return _mlp_tc(sparse, gamma, beta, W1, b1, W2, b2, W3, b3)


# speedup vs baseline: 16.2550x; 1.5665x over previous
import functools

import jax
import jax.numpy as jnp
from jax import lax
from jax.experimental import pallas as pl
from jax.experimental.pallas import tpu as pltpu
from jax.experimental.pallas import tpu_sc as plsc

B, F, L, V, D = 16384, 26, 20, 1000000, 16
H = F * D
BF = B * F
NC, NS = 2, 16
NW = NC * NS
SW = BF // NW
C = 32
CI = C * L
G = CI // 128
CHUNKS = SW // C
IDX_W = SW * L


def _pool_sc(x1d, table):
    mesh = plsc.VectorSubcoreMesh(core_axis_name="c", subcore_axis_name="s")

    @functools.partial(
        pl.kernel,
        out_type=jax.ShapeDtypeStruct((BF, D), jnp.float32),
        mesh=mesh,
        scratch_types=[
            pltpu.VMEM((2, CI), jnp.int32),
            pltpu.VMEM((CI, D), jnp.float32),
            pltpu.VMEM((CI, D), jnp.float32),
            pltpu.VMEM((C, D), jnp.float32),
            pltpu.SemaphoreType.DMA,
            pltpu.SemaphoreType.DMA,
            pltpu.SemaphoreType.DMA,
            pltpu.SemaphoreType.DMA,
        ],
        compiler_params=pltpu.CompilerParams(use_tc_tiling_on_sc=False),
    )
    def pool_kernel(x_hbm, tbl_hbm, out_hbm, idx_v, rows0_v, rows1_v,
                    pooled_v, si0, si1, sg0, sg1):
        wid = lax.axis_index("s") * NC + lax.axis_index("c")
        rows = (rows0_v, rows1_v)
        isem = (si0, si1)
        gsem = (sg0, sg1)

        def fire_idx(slot, g):
            i0 = wid * IDX_W + g * CI
            pltpu.async_copy(x_hbm.at[pl.ds(i0, CI)], idx_v.at[slot],
                             isem[slot])

        def wait_idx(slot):
            pltpu.make_async_copy(x_hbm.at[pl.ds(0, CI)], idx_v.at[slot],
                                  isem[slot]).wait()

        def fire_gather(slot):
            wait_idx(slot)
            for j in range(G):
                pltpu.async_copy(
                    tbl_hbm.at[idx_v.at[slot].at[pl.ds(j * 128, 128)]],
                    rows[slot].at[pl.ds(j * 128, 128)],
                    gsem[slot],
                )

        def wait_gathers(slot):
            for j in range(G):
                pltpu.make_async_copy(
                    tbl_hbm.at[idx_v.at[slot].at[pl.ds(j * 128, 128)]],
                    rows[slot].at[pl.ds(j * 128, 128)],
                    gsem[slot],
                ).wait()

        def acc_out(slot, g):
            @pl.loop(0, C)
            def _seg(s):
                r0 = s * L
                acc = rows[slot][r0]
                for l in range(1, L):
                    acc = acc + rows[slot][r0 + l]
                pooled_v[s] = acc

            pltpu.sync_copy(pooled_v, out_hbm.at[pl.ds(wid * SW + g * C, C)])

        fire_idx(0, 0)
        fire_idx(1, 1)
        fire_gather(0)

        @pl.loop(0, CHUNKS // 2)
        def _pair(gg):
            g0 = gg * 2
            fire_gather(1)
            wait_gathers(0)

            @pl.when(g0 + 2 < CHUNKS)
            def _():
                fire_idx(0, g0 + 2)

            acc_out(0, g0)

            @pl.when(g0 + 2 < CHUNKS)
            def _():
                fire_gather(0)

            wait_gathers(1)

            @pl.when(g0 + 3 < CHUNKS)
            def _():
                fire_idx(1, g0 + 3)

            acc_out(1, g0 + 1)

    return pool_kernel(x1d, table)


BB = 512


def _mlp_tc(sparse, gamma, beta, W1, b1, W2, b2, W3, b3):
    def body(p_ref, g_ref, be_ref, w1_ref, b1_ref, w2_ref, b2_ref, w3_ref,
             b3_ref, o_ref):
        sp = p_ref[...]
        mu = jnp.mean(sp, axis=-1, keepdims=True)
        var = jnp.mean((sp - mu) ** 2, axis=-1, keepdims=True)
        h = (sp - mu) / jnp.sqrt(var + 1e-5) * g_ref[...] + be_ref[...]
        h = jnp.maximum(
            jnp.dot(h, w1_ref[...], preferred_element_type=jnp.float32)
            + b1_ref[...], 0.0)
        h = jnp.maximum(
            jnp.dot(h, w2_ref[...], preferred_element_type=jnp.float32)
            + b2_ref[...], 0.0)
        o = jnp.dot(h, w3_ref[...], preferred_element_type=jnp.float32) + b3_ref[...]
        o_ref[...] = jax.nn.sigmoid(o)

    return pl.pallas_call(
        body,
        grid=(B // BB,),
        in_specs=[
            pl.BlockSpec((BB, H), lambda i: (i, 0)),
            pl.BlockSpec((H,), lambda i: (0,)),
            pl.BlockSpec((H,), lambda i: (0,)),
            pl.BlockSpec((H, 1024), lambda i: (0, 0)),
            pl.BlockSpec((1024,), lambda i: (0,)),
            pl.BlockSpec((1024, 512), lambda i: (0, 0)),
            pl.BlockSpec((512,), lambda i: (0,)),
            pl.BlockSpec((512, 1), lambda i: (0, 0)),
            pl.BlockSpec((1,), lambda i: (0,)),
        ],
        out_specs=pl.BlockSpec((BB, 1), lambda i: (i, 0)),
        out_shape=jax.ShapeDtypeStruct((B, 1), jnp.float32),
    )(sparse, gamma, beta, W1, b1, W2, b2, W3, b3)


def kernel(x, table, gamma, beta, W1, b1, W2, b2, W3, b3):
    x1d = x.reshape(-1)
    pooled = _pool_sc(x1d, table)
    sparse = pooled.reshape(B, H)
    return _mlp_tc(sparse, gamma, beta, W1, b1, W2, b2, W3, b3)


# speedup vs baseline: 16.7186x; 1.0285x over previous
import functools

import jax
import jax.numpy as jnp
from jax import lax
from jax.experimental import pallas as pl
from jax.experimental.pallas import tpu as pltpu
from jax.experimental.pallas import tpu_sc as plsc

B, F, L, V, D = 16384, 26, 20, 1000000, 16
H = F * D
BF = B * F
NC, NS = 2, 16
NW = NC * NS
SW = BF // NW
C = 32
CI = C * L
G = CI // 128
CHUNKS = SW // C
IDX_W = SW * L


def _pool_sc(x1d, table):
    mesh = plsc.VectorSubcoreMesh(core_axis_name="c", subcore_axis_name="s")

    @functools.partial(
        pl.kernel,
        out_type=jax.ShapeDtypeStruct((BF, D), jnp.float32),
        mesh=mesh,
        scratch_types=[
            pltpu.VMEM((2, CI), jnp.int32),
            pltpu.VMEM((CI, D), jnp.float32),
            pltpu.VMEM((CI, D), jnp.float32),
            pltpu.VMEM((C, D), jnp.float32),
            pltpu.SemaphoreType.DMA,
            pltpu.SemaphoreType.DMA,
            pltpu.SemaphoreType.DMA,
            pltpu.SemaphoreType.DMA,
        ],
        compiler_params=pltpu.CompilerParams(use_tc_tiling_on_sc=False),
    )
    def pool_kernel(x_hbm, tbl_hbm, out_hbm, idx_v, rows0_v, rows1_v,
                    pooled_v, si0, si1, sg0, sg1):
        wid = lax.axis_index("s") * NC + lax.axis_index("c")
        rows = (rows0_v, rows1_v)
        isem = (si0, si1)
        gsem = (sg0, sg1)

        def fire_idx(slot, g):
            i0 = wid * IDX_W + g * CI
            pltpu.async_copy(x_hbm.at[pl.ds(i0, CI)], idx_v.at[slot],
                             isem[slot])

        def wait_idx(slot):
            pltpu.make_async_copy(x_hbm.at[pl.ds(0, CI)], idx_v.at[slot],
                                  isem[slot]).wait()

        def fire_gather(slot):
            wait_idx(slot)
            for j in range(G):
                pltpu.async_copy(
                    tbl_hbm.at[idx_v.at[slot].at[pl.ds(j * 128, 128)]],
                    rows[slot].at[pl.ds(j * 128, 128)],
                    gsem[slot],
                )

        def wait_gathers(slot):
            for j in range(G):
                pltpu.make_async_copy(
                    tbl_hbm.at[idx_v.at[slot].at[pl.ds(j * 128, 128)]],
                    rows[slot].at[pl.ds(j * 128, 128)],
                    gsem[slot],
                ).wait()

        def acc_out(slot, g):
            @pl.loop(0, C)
            def _seg(s):
                r0 = s * L
                vals = [rows[slot][r0 + l] for l in range(L)]
                while len(vals) > 1:
                    nxt = [vals[i] + vals[i + 1]
                           for i in range(0, len(vals) - 1, 2)]
                    if len(vals) % 2:
                        nxt.append(vals[-1])
                    vals = nxt
                pooled_v[s] = vals[0]

            pltpu.sync_copy(pooled_v, out_hbm.at[pl.ds(wid * SW + g * C, C)])

        fire_idx(0, 0)
        fire_idx(1, 1)
        fire_gather(0)

        @pl.loop(0, CHUNKS // 2)
        def _pair(gg):
            g0 = gg * 2
            fire_gather(1)
            wait_gathers(0)

            @pl.when(g0 + 2 < CHUNKS)
            def _():
                fire_idx(0, g0 + 2)

            acc_out(0, g0)

            @pl.when(g0 + 2 < CHUNKS)
            def _():
                fire_gather(0)

            wait_gathers(1)

            @pl.when(g0 + 3 < CHUNKS)
            def _():
                fire_idx(1, g0 + 3)

            acc_out(1, g0 + 1)

    return pool_kernel(x1d, table)


BB = 512


def _mlp_tc(sparse, gamma, beta, W1, b1, W2, b2, W3, b3):
    def body(p_ref, g_ref, be_ref, w1_ref, b1_ref, w2_ref, b2_ref, w3_ref,
             b3_ref, o_ref):
        sp = p_ref[...]
        mu = jnp.mean(sp, axis=-1, keepdims=True)
        var = jnp.mean((sp - mu) ** 2, axis=-1, keepdims=True)
        h = (sp - mu) / jnp.sqrt(var + 1e-5) * g_ref[...] + be_ref[...]
        h = jnp.maximum(
            jnp.dot(h.astype(jnp.bfloat16), w1_ref[...].astype(jnp.bfloat16),
                    preferred_element_type=jnp.float32)
            + b1_ref[...], 0.0)
        h = jnp.maximum(
            jnp.dot(h.astype(jnp.bfloat16), w2_ref[...].astype(jnp.bfloat16),
                    preferred_element_type=jnp.float32)
            + b2_ref[...], 0.0)
        o = jnp.dot(h, w3_ref[...], preferred_element_type=jnp.float32) + b3_ref[...]
        o_ref[...] = jax.nn.sigmoid(o)

    return pl.pallas_call(
        body,
        grid=(B // BB,),
        in_specs=[
            pl.BlockSpec((BB, H), lambda i: (i, 0)),
            pl.BlockSpec((H,), lambda i: (0,)),
            pl.BlockSpec((H,), lambda i: (0,)),
            pl.BlockSpec((H, 1024), lambda i: (0, 0)),
            pl.BlockSpec((1024,), lambda i: (0,)),
            pl.BlockSpec((1024, 512), lambda i: (0, 0)),
            pl.BlockSpec((512,), lambda i: (0,)),
            pl.BlockSpec((512, 1), lambda i: (0, 0)),
            pl.BlockSpec((1,), lambda i: (0,)),
        ],
        out_specs=pl.BlockSpec((BB, 1), lambda i: (i, 0)),
        out_shape=jax.ShapeDtypeStruct((B, 1), jnp.float32),
    )(sparse, gamma, beta, W1, b1, W2, b2, W3, b3)


def kernel(x, table, gamma, beta, W1, b1, W2, b2, W3, b3):
    x1d = x.reshape(-1)
    pooled = _pool_sc(x1d, table)
    sparse = pooled.reshape(B, H)
    return _mlp_tc(sparse, gamma, beta, W1, b1, W2, b2, W3, b3)
